# Initial kernel scaffold; baseline (speedup 1.0000x reference)
import jax
import jax.numpy as jnp
from jax.experimental import pallas as pl


def kernel(A, x):
    raise NotImplementedError("write your pallas kernel here")


# speedup vs baseline: 1.0046x; 1.0046x over previous
import jax
import jax.numpy as jnp
from jax.experimental import pallas as pl

_BM = 512
_BK = 4096


def _mv_block(a_ref, x_ref, o_ref):
    j = pl.program_id(1)
    partial = jax.lax.dot_general(
        a_ref[...], x_ref[...],
        dimension_numbers=(((1,), (1,)), ((), ())),
        preferred_element_type=jnp.float32,
    )

    @pl.when(j == 0)
    def _init():
        o_ref[...] = partial

    @pl.when(j != 0)
    def _acc():
        o_ref[...] += partial


def kernel(A, x):
    m, k = A.shape
    x2 = x.reshape(1, k)
    out = pl.pallas_call(
        _mv_block,
        grid=(m // _BM, k // _BK),
        in_specs=[
            pl.BlockSpec((_BM, _BK), lambda i, j: (i, j)),
            pl.BlockSpec((1, _BK), lambda i, j: (0, j)),
        ],
        out_specs=pl.BlockSpec((_BM, 1), lambda i, j: (i, 0)),
        out_shape=jax.ShapeDtypeStruct((m, 1), jnp.float32),
    )(A, x2)
    return out.reshape(m)


# speedup vs baseline: 1.0056x; 1.0010x over previous
import jax
import jax.numpy as jnp
from jax.experimental import pallas as pl

_BM = 256


def _mv_block(a_ref, x_ref, o_ref):
    o_ref[...] = jax.lax.dot_general(
        a_ref[...], x_ref[...],
        dimension_numbers=(((1,), (1,)), ((), ())),
        preferred_element_type=jnp.float32,
    )


def kernel(A, x):
    m, k = A.shape
    x2 = x.reshape(1, k)
    out = pl.pallas_call(
        _mv_block,
        grid=(m // _BM,),
        in_specs=[
            pl.BlockSpec((_BM, k), lambda i: (i, 0)),
            pl.BlockSpec((1, k), lambda i: (0, 0)),
        ],
        out_specs=pl.BlockSpec((_BM, 1), lambda i: (i, 0)),
        out_shape=jax.ShapeDtypeStruct((m, 1), jnp.float32),
    )(A, x2)
    return out.reshape(m)
